# Initial kernel scaffold; baseline (speedup 1.0000x reference)
import functools

import jax
import jax.numpy as jnp
from jax import lax
from jax.experimental import pallas as pl
from jax.experimental.pallas import tpu as pltpu
from jax.experimental.pallas import tpu_sc as plsc

N_ATOMS = 3200000
N_PROPS = 14
TAB_PAD = 96
CHUNK = 2000
GSZ = 80
NGATHER = CHUNK // GSZ
L = 16


def _make_kernel(nc, ns):
    nw = nc * ns
    per_w = N_ATOMS // nw
    n_chunks = per_w // CHUNK
    mesh = plsc.VectorSubcoreMesh(core_axis_name="c", subcore_axis_name="s")

    @functools.partial(
        pl.kernel,
        mesh=mesh,
        out_type=(
            jax.ShapeDtypeStruct((N_ATOMS,), jnp.int32),
            jax.ShapeDtypeStruct((N_ATOMS,), jnp.int32),
            jax.ShapeDtypeStruct((N_ATOMS, N_PROPS), jnp.float32),
        ),
        scratch_types=[
            pltpu.VMEM((CHUNK,), jnp.int32),
            pltpu.VMEM((CHUNK,), jnp.int32),
            pltpu.VMEM((CHUNK,), jnp.int32),
            pltpu.VMEM((CHUNK, N_PROPS), jnp.float32),
            pltpu.VMEM((TAB_PAD,), jnp.int32),
            pltpu.VMEM((TAB_PAD,), jnp.int32),
            pltpu.SemaphoreType.DMA,
        ],
    )
    def phys_ref_sc(z_hbm, ptab_hbm, gtab_hbm, props_hbm,
                    period_out, group_out, props_out,
                    z_v, per_v, grp_v, rows_v, ptab_v, gtab_v, sem):
        wid = lax.axis_index("s") * nc + lax.axis_index("c")
        base = wid * per_w

        pltpu.sync_copy(ptab_hbm, ptab_v)
        pltpu.sync_copy(gtab_hbm, gtab_v)

        def step(i, carry):
            off = base + i * CHUNK
            pltpu.sync_copy(z_hbm.at[pl.ds(off, CHUNK)], z_v)

            copies = []
            for k in range(NGATHER):
                copies.append(pltpu.async_copy(
                    props_hbm.at[z_v.at[pl.ds(k * GSZ, GSZ)]],
                    rows_v.at[pl.ds(k * GSZ, GSZ)],
                    sem,
                ))

            def grp(j, c):
                zv = z_v[pl.ds(j * L, L)]
                per_v[pl.ds(j * L, L)] = plsc.load_gather(ptab_v, [zv])
                grp_v[pl.ds(j * L, L)] = plsc.load_gather(gtab_v, [zv])
                return c
            lax.fori_loop(0, CHUNK // L, grp, 0)

            for cp in copies:
                cp.wait()

            pltpu.sync_copy(per_v, period_out.at[pl.ds(off, CHUNK)])
            pltpu.sync_copy(grp_v, group_out.at[pl.ds(off, CHUNK)])
            pltpu.sync_copy(rows_v, props_out.at[pl.ds(off, CHUNK)])
            return carry

        lax.fori_loop(0, n_chunks, step, 0)

    return phys_ref_sc


def kernel(z, period_mapping, group_mapping, properties_mapping):
    info = plsc.get_sparse_core_info()
    f = _make_kernel(info.num_cores, info.num_subcores)
    pad = TAB_PAD - period_mapping.shape[0]
    ptab = jnp.pad(period_mapping.astype(jnp.int32), (0, pad))
    gtab = jnp.pad(group_mapping.astype(jnp.int32), (0, pad))
    period, group, props = f(z, ptab, gtab, properties_mapping)
    return (period, group, props)


# speedup vs baseline: 19.5049x; 19.5049x over previous
import functools

import jax
import jax.numpy as jnp
from jax import lax
from jax.experimental import pallas as pl
from jax.experimental.pallas import tpu as pltpu
from jax.experimental.pallas import tpu_sc as plsc

N_ATOMS = 3200000
N_PROPS = 14
ROW_PAD = 16
TAB_PAD = 96
GSZ = 80
NG = 10
CHUNK = NG * GSZ
L = 16
UNROLL = 8


def _make_kernel(nc, ns):
    nw = nc * ns
    per_w = N_ATOMS // nw
    n_chunks = per_w // CHUNK
    assert per_w % CHUNK == 0
    mesh = plsc.VectorSubcoreMesh(core_axis_name="c", subcore_axis_name="s")

    @functools.partial(
        pl.kernel,
        mesh=mesh,
        compiler_params=pltpu.CompilerParams(needs_layout_passes=False,
                                             use_tc_tiling_on_sc=False),
        out_type=(
            jax.ShapeDtypeStruct((N_ATOMS,), jnp.int32),
            jax.ShapeDtypeStruct((N_ATOMS,), jnp.int32),
            jax.ShapeDtypeStruct((N_ATOMS * N_PROPS,), jnp.float32),
        ),
        scratch_types=[
            pltpu.VMEM((NG, GSZ), jnp.int32),
            pltpu.VMEM((CHUNK,), jnp.int32),
            pltpu.VMEM((CHUNK,), jnp.int32),
            pltpu.VMEM((CHUNK, ROW_PAD), jnp.float32),
            pltpu.VMEM((CHUNK * N_PROPS + 2,), jnp.float32),
            pltpu.VMEM((TAB_PAD,), jnp.int32),
            pltpu.VMEM((TAB_PAD,), jnp.int32),
            pltpu.SemaphoreType.DMA,
        ],
    )
    def phys_ref_sc(z2_hbm, ptab_hbm, gtab_hbm, props_hbm,
                    period_out, group_out, props_out,
                    z_v, per_v, grp_v, rows_v, flat_v, ptab_v, gtab_v, sem):
        wid = lax.axis_index("s") * nc + lax.axis_index("c")
        row_base = wid * (per_w // GSZ)

        pltpu.sync_copy(ptab_hbm, ptab_v)
        pltpu.sync_copy(gtab_hbm, gtab_v)

        def step(i, carry):
            row_off = row_base + i * NG
            off = row_off * GSZ
            pltpu.sync_copy(z2_hbm.at[pl.ds(row_off, NG)], z_v)

            copies = []
            for k in range(NG):
                copies.append(pltpu.async_copy(
                    props_hbm.at[z_v.at[k]],
                    rows_v.at[pl.ds(k * GSZ, GSZ)], sem))

            def grp(t, c):
                k = t // (GSZ // L)
                j = t % (GSZ // L)
                zv = z_v[k, pl.ds(j * L, L)]
                per_v[pl.ds(t * L, L)] = plsc.load_gather(ptab_v, [zv])
                grp_v[pl.ds(t * L, L)] = plsc.load_gather(gtab_v, [zv])
                return c
            lax.fori_loop(0, CHUNK // L, grp, 0)

            for cp in copies:
                cp.wait()

            def compact(t, c):
                for u in range(UNROLL):
                    a = t * UNROLL + u
                    flat_v[pl.ds(a * N_PROPS, L)] = rows_v[a, :]
                return c
            lax.fori_loop(0, CHUNK // UNROLL, compact, 0)

            pltpu.sync_copy(per_v, period_out.at[pl.ds(off, CHUNK)])
            pltpu.sync_copy(grp_v, group_out.at[pl.ds(off, CHUNK)])
            pltpu.sync_copy(flat_v.at[pl.ds(0, CHUNK * N_PROPS)],
                            props_out.at[pl.ds(off * N_PROPS,
                                               CHUNK * N_PROPS)])
            return carry

        lax.fori_loop(0, n_chunks, step, 0)

    return phys_ref_sc


def kernel(z, period_mapping, group_mapping, properties_mapping):
    info = plsc.get_sparse_core_info()
    f = _make_kernel(info.num_cores, info.num_subcores)
    pad = TAB_PAD - period_mapping.shape[0]
    ptab = jnp.pad(period_mapping.astype(jnp.int32), (0, pad))
    gtab = jnp.pad(group_mapping.astype(jnp.int32), (0, pad))
    props_pad = jnp.pad(properties_mapping, ((0, 0), (0, ROW_PAD - N_PROPS)))
    z2 = z.reshape(N_ATOMS // GSZ, GSZ)
    period, group, props_flat = f(z2, ptab, gtab, props_pad)
    return (period, group, props_flat.reshape(N_ATOMS, N_PROPS))


# speedup vs baseline: 48.0178x; 2.4618x over previous
import functools

import jax
import jax.numpy as jnp
from jax import lax
from jax.experimental import pallas as pl
from jax.experimental.pallas import tpu as pltpu
from jax.experimental.pallas import tpu_sc as plsc

N_ATOMS = 3200000
N_PROPS = 14
ROW_PAD = 16
TAB_PAD = 96
BLK = 128
NBLK = N_ATOMS // BLK
CB = 8
CHUNK = CB * BLK
NCHUNK = NBLK // CB
L = 16


def _make_kernel(nc, ns):
    nw = nc * ns
    per_w = -(-NCHUNK // nw)
    mesh = plsc.VectorSubcoreMesh(core_axis_name="c", subcore_axis_name="s")

    @functools.partial(
        pl.kernel,
        mesh=mesh,
        compiler_params=pltpu.CompilerParams(needs_layout_passes=False,
                                             use_tc_tiling_on_sc=False),
        out_type=(
            jax.ShapeDtypeStruct((N_ATOMS,), jnp.int32),
            jax.ShapeDtypeStruct((N_ATOMS,), jnp.int32),
            jax.ShapeDtypeStruct((2, NBLK, 8, BLK), jnp.float32),
        ),
        scratch_types=[
            pltpu.VMEM((CB, BLK), jnp.int32),
            pltpu.VMEM((CHUNK,), jnp.int32),
            pltpu.VMEM((CHUNK,), jnp.int32),
            pltpu.VMEM((CHUNK, ROW_PAD), jnp.float32),
            pltpu.VMEM((CB, 8, BLK), jnp.float32),
            pltpu.VMEM((CB, 8, BLK), jnp.float32),
            pltpu.VMEM((TAB_PAD,), jnp.int32),
            pltpu.VMEM((TAB_PAD,), jnp.int32),
            pltpu.SemaphoreType.DMA,
        ],
    )
    def phys_ref_sc(z2_hbm, ptab_hbm, gtab_hbm, props_hbm,
                    period_out, group_out, props_out,
                    z_v, per_v, grp_v, rows_v, p0_v, p1_v,
                    ptab_v, gtab_v, sem):
        wid = lax.axis_index("s") * nc + lax.axis_index("c")
        start = wid * per_w
        count = jnp.clip(NCHUNK - start, 0, per_w)

        pltpu.sync_copy(ptab_hbm, ptab_v)
        pltpu.sync_copy(gtab_hbm, gtab_v)

        lane = lax.iota(jnp.int32, L)

        def step(i, carry):
            g = start + i
            b0 = g * CB
            pltpu.sync_copy(z2_hbm.at[pl.ds(b0, CB)], z_v)

            copies = []
            for k in range(CB):
                copies.append(pltpu.async_copy(
                    props_hbm.at[z_v.at[k]],
                    rows_v.at[pl.ds(k * BLK, BLK)], sem))

            def grp(t, c):
                blk = t // (BLK // L)
                gg = t % (BLK // L)
                zv = z_v[blk, pl.ds(gg * L, L)]
                per_v[pl.ds(t * L, L)] = plsc.load_gather(ptab_v, [zv])
                grp_v[pl.ds(t * L, L)] = plsc.load_gather(gtab_v, [zv])
                return c
            lax.fori_loop(0, CHUNK // L, grp, 0)

            for cp in copies:
                cp.wait()

            def tr(t, c):
                blk = t // (BLK // L)
                gg = t % (BLK // L)
                row_vec = blk * BLK + gg * L + lane
                for j in range(N_PROPS):
                    v = plsc.load_gather(
                        rows_v, [row_vec, jnp.full((L,), j, jnp.int32)])
                    if j < 8:
                        p0_v[blk, j, pl.ds(gg * L, L)] = v
                    else:
                        p1_v[blk, j - 8, pl.ds(gg * L, L)] = v
                return c
            lax.fori_loop(0, CHUNK // L, tr, 0)

            pltpu.sync_copy(per_v, period_out.at[pl.ds(g * CHUNK, CHUNK)])
            pltpu.sync_copy(grp_v, group_out.at[pl.ds(g * CHUNK, CHUNK)])
            pltpu.sync_copy(p0_v, props_out.at[0, pl.ds(b0, CB)])
            pltpu.sync_copy(p1_v, props_out.at[1, pl.ds(b0, CB)])
            return carry

        lax.fori_loop(0, count, step, 0)

    return phys_ref_sc


def kernel(z, period_mapping, group_mapping, properties_mapping):
    info = plsc.get_sparse_core_info()
    f = _make_kernel(info.num_cores, info.num_subcores)
    pad = TAB_PAD - period_mapping.shape[0]
    ptab = jnp.pad(period_mapping.astype(jnp.int32), (0, pad))
    gtab = jnp.pad(group_mapping.astype(jnp.int32), (0, pad))
    props_pad = jnp.pad(properties_mapping, ((0, 0), (0, ROW_PAD - N_PROPS)))
    z2 = z.reshape(NBLK, BLK)
    period, group, planes = f(z2, ptab, gtab, props_pad)
    props = planes.transpose(1, 3, 0, 2).reshape(N_ATOMS, 16)[:, :N_PROPS]
    return (period, group, props)
